# baseline (speedup 1.0000x reference)
import functools

import jax
import jax.numpy as jnp
import numpy as np
from jax import lax
from jax.experimental import pallas as pl
from jax.experimental.pallas import tpu as pltpu
from jax.experimental.pallas import tpu_sc as plsc

F = 26
VOCAB = 100000
EMB = 16
BATCH = 16384
D1 = 32
D2 = 32

ROWS = BATCH * F
NC, NS = 2, 16
NW = NC * NS
RPW = ROWS // NW
CH = 1664
NCH = RPW // CH
BPC = CH // F
FOUT_RPC = BPC * 32 // 128


CH32 = BPC * 32


def _gather_body(sec_hbm, first_hbm, idxf_hbm, idx32_hbm,
                 sec_out, first_out,
                 idx_v, rows_v, rv128, idx32_v, f1_v, f2_v, sem, fsem):
    wid = lax.axis_index("s") * NC + lax.axis_index("c")
    base = wid * RPW

    def rep_sec(j, carry):
        for k in range(8):
            rv128[j, pl.ds(k * 16, 16)] = rows_v[8 * j + k, :]
        return carry

    def rep_first(r, carry):
        for s in range(8):
            f2_v[r, pl.ds(s * 16, 16)] = f1_v[pl.ds(r * 128 + s * 16, 16)]
        return carry

    for c in range(NCH):
        off = base + c * CH
        b0 = off // F
        pltpu.sync_copy(idxf_hbm.at[pl.ds(off, CH)], idx_v)
        pltpu.async_copy(sec_hbm.at[idx_v], rows_v, sem).wait()
        lax.fori_loop(0, CH // 8, rep_sec, 0)
        pltpu.sync_copy(rv128, sec_out.at[pl.ds(off // 8, CH // 8)])

        pltpu.sync_copy(idx32_hbm.at[pl.ds(b0 * 32, CH32)], idx32_v)
        pltpu.async_copy(first_hbm.at[idx32_v], f1_v, fsem).wait()
        lax.fori_loop(0, FOUT_RPC, rep_first, 0)
        pltpu.sync_copy(f2_v, first_out.at[pl.ds(b0 // 4, FOUT_RPC)])


@functools.lru_cache(maxsize=None)
def _make_gather():
    return pl.kernel(
        _gather_body,
        mesh=plsc.VectorSubcoreMesh(core_axis_name="c", subcore_axis_name="s"),
        compiler_params=pltpu.CompilerParams(use_tc_tiling_on_sc=False),
        out_type=(
            jax.ShapeDtypeStruct((ROWS // 8, 128), jnp.float32),
            jax.ShapeDtypeStruct((BATCH * 32 // 128, 128), jnp.float32),
        ),
        scratch_types=[
            pltpu.VMEM((CH,), jnp.int32),
            pltpu.VMEM((CH, EMB), jnp.float32),
            pltpu.VMEM((CH // 8, 128), jnp.float32),
            pltpu.VMEM((CH32,), jnp.int32),
            pltpu.VMEM((CH32,), jnp.float32),
            pltpu.VMEM((FOUT_RPC, 128), jnp.float32),
            pltpu.SemaphoreType.DMA,
            pltpu.SemaphoreType.DMA,
        ],
    )


BS = 4096
NBLK = BATCH // BS
PR = BS // 8


def _mlp_body(sec8_ref, xvp8_ref, first4_ref, xvexp4_ref, w1bblk_ref,
              w1ablk_ref, b1_ref, w2_ref, b2_ref, w3_ref, b3_ref,
              out_ref, acc):
    f = pl.program_id(1)

    @pl.when(f == 0)
    def _():
        acc[:, :] = jnp.zeros_like(acc)

    xs = sec8_ref[:, :] * xvp8_ref[:, :]
    acc[:, :] = acc[:, :] + jnp.dot(xs, w1bblk_ref[0],
                                    preferred_element_type=jnp.float32)

    @pl.when(f == F - 1)
    def _():
        fo4 = first4_ref[:, :] * xvexp4_ref[:, :]
        c4 = jnp.dot(fo4, w1ablk_ref[:, :],
                     preferred_element_type=jnp.float32)
        first8 = c4.reshape(PR, 256)
        h = jnp.maximum(acc[:, :] + first8 + b1_ref[:, :], 0.0)
        h = jnp.maximum(
            jnp.dot(h, w2_ref[:, :], preferred_element_type=jnp.float32)
            + b2_ref[:, :], 0.0)
        out_ref[:, :] = (
            jnp.dot(h, w3_ref[:, :], preferred_element_type=jnp.float32)
            + b3_ref[:, :])


def _mlp(sec8, xvp8, first4, xvexp4, w1bblk, w1ablk, b1blk, w2blk, b2blk,
         w3blk, b3blk):
    zero2 = lambda i, f: (0, 0)
    return pl.pallas_call(
        _mlp_body,
        grid=(NBLK, F),
        in_specs=[
            pl.BlockSpec((PR, 128), lambda i, f: (f * NBLK + i, 0)),
            pl.BlockSpec((PR, 128), lambda i, f: (f * NBLK + i, 0)),
            pl.BlockSpec((BS // 4, 128), lambda i, f: (i, 0)),
            pl.BlockSpec((BS // 4, 128), lambda i, f: (i, 0)),
            pl.BlockSpec((1, 128, 256), lambda i, f: (f, 0, 0)),
            pl.BlockSpec((128, 128), zero2),
            pl.BlockSpec((1, 256), zero2),
            pl.BlockSpec((256, 256), zero2),
            pl.BlockSpec((1, 256), zero2),
            pl.BlockSpec((256, 8), zero2),
            pl.BlockSpec((1, 8), zero2),
        ],
        out_specs=pl.BlockSpec((PR, 8), lambda i, f: (i, 0)),
        out_shape=jax.ShapeDtypeStruct((BATCH // 8, 8), jnp.float32),
        scratch_shapes=[pltpu.VMEM((PR, 256), jnp.float32)],
    )(sec8, xvp8, first4, xvexp4, w1bblk, w1ablk, b1blk, w2blk, b2blk,
      w3blk, b3blk)


def kernel(Xi, Xv, fm_bias, first_tables, second_tables, W1, b1, W2, b2, W3, b3):
    xi = Xi[:, :, 0].astype(jnp.int32)
    xv = Xv.astype(jnp.float32)
    foff = jnp.arange(F, dtype=jnp.int32) * VOCAB
    idx_f = (xi.T + foff[:, None]).reshape(ROWS)
    idx32 = jnp.pad(xi + foff[None, :], ((0, 0), (0, 6))).reshape(
        BATCH * 32)
    sec_flat = second_tables.reshape(F * VOCAB, EMB)
    first_flat = first_tables.reshape(F * VOCAB)

    sec8, first4 = _make_gather()(sec_flat, first_flat, idx_f, idx32)

    xvp8 = jnp.broadcast_to(xv.T[:, :, None], (F, BATCH, EMB)).reshape(
        ROWS * EMB // 128, 128)
    xvexp4 = jnp.pad(xv, ((0, 0), (0, 6))).reshape(BATCH * 32 // 128, 128)

    eye8 = jnp.eye(8, dtype=jnp.float32)
    w1b3 = W1[1 + F:, :].reshape(F, EMB, D1)
    w1bblk = (eye8[None, :, None, :, None]
              * w1b3[:, None, :, None, :]).reshape(F, 128, 8 * D1)
    w1apad = jnp.pad(W1[1:1 + F, :], ((0, 6), (0, 0)))
    w1ablk = jnp.kron(jnp.eye(4, dtype=jnp.float32), w1apad)
    b1e = (b1 + fm_bias * W1[0, :]).reshape(1, D1)
    b1blk = jnp.tile(b1e, (1, 8))
    w2blk = jnp.kron(eye8, W2)
    b2blk = jnp.tile(b2.reshape(1, D2), (1, 8))
    w3blk = jnp.kron(eye8, W3)
    b3blk = jnp.tile(b3.reshape(1, 1), (1, 8))

    out8 = _mlp(sec8, xvp8, first4, xvexp4, w1bblk, w1ablk, b1blk,
                w2blk, b2blk, w3blk, b3blk)
    return out8.reshape(BATCH)


# speedup vs baseline: 1.0810x; 1.0810x over previous
import functools

import jax
import jax.numpy as jnp
import numpy as np
from jax import lax
from jax.experimental import pallas as pl
from jax.experimental.pallas import tpu as pltpu
from jax.experimental.pallas import tpu_sc as plsc

F = 26
VOCAB = 100000
EMB = 16
BATCH = 16384
D1 = 32
D2 = 32

ROWS = BATCH * F
NC, NS = 2, 16
NW = NC * NS
RPW = ROWS // NW
CH = 1664
NCH = RPW // CH
BPC = CH // F
FOUT_RPC = BPC * 32 // 128


CH32 = BPC * 32
CHR = CH // 8


def _gather_body(sec_hbm, first_hbm, idxf_hbm, idx32_hbm,
                 sec_out, first_out,
                 idx_v, rows_v, idx32_v, f1_v, f2_v, sem, fsem):
    wid = lax.axis_index("s") * NC + lax.axis_index("c")
    base = wid * RPW
    for c in range(NCH):
        off = base + c * CH
        b0 = off // F
        pltpu.sync_copy(idxf_hbm.at[pl.ds(off, CH)], idx_v)
        gs = [pltpu.async_copy(
                  sec_hbm.at[idx_v.at[pl.ds(k * CHR, CHR)]],
                  rows_v.at[k], sem) for k in range(8)]
        for g in gs:
            g.wait()
        for k in range(8):
            pltpu.sync_copy(
                rows_v.at[k],
                sec_out.at[pl.ds(off // 8, CHR), pl.ds(k * 16, 16)])

        pltpu.sync_copy(idx32_hbm.at[pl.ds(b0 * 32, CH32)], idx32_v)
        pltpu.async_copy(first_hbm.at[idx32_v], f1_v, fsem).wait()
        for r in range(FOUT_RPC):
            for sg in range(8):
                f2_v[r, pl.ds(sg * 16, 16)] = (
                    f1_v[pl.ds(r * 128 + sg * 16, 16)])
        pltpu.sync_copy(f2_v, first_out.at[pl.ds(b0 // 4, FOUT_RPC)])


@functools.lru_cache(maxsize=None)
def _make_gather():
    return pl.kernel(
        _gather_body,
        mesh=plsc.VectorSubcoreMesh(core_axis_name="c", subcore_axis_name="s"),
        compiler_params=pltpu.CompilerParams(use_tc_tiling_on_sc=False),
        out_type=(
            jax.ShapeDtypeStruct((ROWS // 8, 128), jnp.float32),
            jax.ShapeDtypeStruct((BATCH * 32 // 128, 128), jnp.float32),
        ),
        scratch_types=[
            pltpu.VMEM((CH,), jnp.int32),
            pltpu.VMEM((8, CHR, EMB), jnp.float32),
            pltpu.VMEM((CH32,), jnp.int32),
            pltpu.VMEM((CH32,), jnp.float32),
            pltpu.VMEM((FOUT_RPC, 128), jnp.float32),
            pltpu.SemaphoreType.DMA,
            pltpu.SemaphoreType.DMA,
        ],
    )


BS = 4096
NBLK = BATCH // BS
PR = BS // 8


def _mlp_body(sec8_ref, first4_ref, xv_ref, w1bblk_ref,
              w1ablk_ref, s32_ref, p512_ref, b1_ref, w2_ref, b2_ref,
              w3_ref, b3_ref, out_ref, acc, xv4_s):
    f = pl.program_id(1)

    @pl.when(f == 0)
    def _():
        acc[:, :] = jnp.zeros_like(acc)
        xvwide = jnp.concatenate(
            [xv_ref[:, :], jnp.zeros((BS, 128 - F), jnp.float32)], axis=1)
        xvb = xvwide.reshape(BS // 2, 256).reshape(BS // 4, 512)
        xv4_s[:, :] = jnp.dot(xvb, p512_ref[:, :],
                              preferred_element_type=jnp.float32)

    xv4all = xv4_s[:, :]
    lane128 = lax.broadcasted_iota(jnp.int32, (BS // 4, 128), 1)
    xm = jnp.where(lane128 % 32 == f, xv4all, 0.0)
    u = jnp.dot(xm, s32_ref[:, :],
                preferred_element_type=jnp.float32)
    xvq = u.reshape(PR, 256)
    q = jnp.dot(sec8_ref[:, :], w1bblk_ref[0],
                preferred_element_type=jnp.float32)
    acc[:, :] = acc[:, :] + q * xvq

    @pl.when(f == F - 1)
    def _():
        fo4 = first4_ref[:, :] * xv4all
        c4 = jnp.dot(fo4, w1ablk_ref[:, :],
                     preferred_element_type=jnp.float32)
        first8 = c4.reshape(PR, 256)
        h = jnp.maximum(acc[:, :] + first8 + b1_ref[:, :], 0.0)
        h = jnp.maximum(
            jnp.dot(h, w2_ref[:, :], preferred_element_type=jnp.float32)
            + b2_ref[:, :], 0.0)
        out_ref[:, :] = (
            jnp.dot(h, w3_ref[:, :], preferred_element_type=jnp.float32)
            + b3_ref[:, :])


def _mlp(sec8, first4, xv, w1bblk, w1ablk, s32, p512, b1blk, w2blk, b2blk,
         w3blk, b3blk):
    zero2 = lambda i, f: (0, 0)
    return pl.pallas_call(
        _mlp_body,
        grid=(NBLK, F),
        in_specs=[
            pl.BlockSpec((PR, 128), lambda i, f: (f * NBLK + i, 0)),
            pl.BlockSpec((BS // 4, 128), lambda i, f: (i, 0)),
            pl.BlockSpec((BS, F), lambda i, f: (i, 0)),
            pl.BlockSpec((1, 128, 256), lambda i, f: (f, 0, 0)),
            pl.BlockSpec((128, 128), zero2),
            pl.BlockSpec((128, 128), zero2),
            pl.BlockSpec((512, 128), zero2),
            pl.BlockSpec((1, 256), zero2),
            pl.BlockSpec((256, 256), zero2),
            pl.BlockSpec((1, 256), zero2),
            pl.BlockSpec((256, 8), zero2),
            pl.BlockSpec((1, 8), zero2),
        ],
        out_specs=pl.BlockSpec((PR, 8), lambda i, f: (i, 0)),
        out_shape=jax.ShapeDtypeStruct((BATCH // 8, 8), jnp.float32),
        scratch_shapes=[pltpu.VMEM((PR, 256), jnp.float32),
                        pltpu.VMEM((BS // 4, 128), jnp.float32)],
    )(sec8, first4, xv, w1bblk, w1ablk, s32, p512, b1blk, w2blk, b2blk,
      w3blk, b3blk)


def kernel(Xi, Xv, fm_bias, first_tables, second_tables, W1, b1, W2, b2, W3, b3):
    xi = Xi[:, :, 0].astype(jnp.int32)
    xv = Xv.astype(jnp.float32)
    foff = jnp.arange(F, dtype=jnp.int32) * VOCAB
    idx_f = (xi.T + foff[:, None]).reshape(ROWS)
    idx_f = idx_f.reshape(NW * NCH, CHR, 8).swapaxes(1, 2).reshape(ROWS)
    idx32 = jnp.pad(xi + foff[None, :], ((0, 0), (0, 6))).reshape(
        BATCH * 32)
    sec_flat = second_tables.reshape(F * VOCAB, EMB)
    first_flat = first_tables.reshape(F * VOCAB)

    sec8, first4 = _make_gather()(sec_flat, first_flat, idx_f, idx32)

    eye8 = jnp.eye(8, dtype=jnp.float32)
    s32 = jnp.kron(jnp.eye(4, dtype=jnp.float32),
                   jnp.ones((32, 32), jnp.float32))
    p512np = np.zeros((512, 128), np.float32)
    for k in range(4):
        for j in range(32):
            p512np[k * 128 + j, k * 32 + j] = 1.0
    p512 = jnp.asarray(p512np)
    w1b3 = W1[1 + F:, :].reshape(F, EMB, D1)
    w1bblk = (eye8[None, :, None, :, None]
              * w1b3[:, None, :, None, :]).reshape(F, 128, 8 * D1)
    w1apad = jnp.pad(W1[1:1 + F, :], ((0, 6), (0, 0)))
    w1ablk = jnp.kron(jnp.eye(4, dtype=jnp.float32), w1apad)
    b1e = (b1 + fm_bias * W1[0, :]).reshape(1, D1)
    b1blk = jnp.tile(b1e, (1, 8))
    w2blk = jnp.kron(eye8, W2)
    b2blk = jnp.tile(b2.reshape(1, D2), (1, 8))
    w3blk = jnp.kron(eye8, W3)
    b3blk = jnp.tile(b3.reshape(1, 1), (1, 8))

    out8 = _mlp(sec8, first4, xv, w1bblk, w1ablk, s32, p512, b1blk,
                w2blk, b2blk, w3blk, b3blk)
    return out8.reshape(BATCH)


# speedup vs baseline: 1.0825x; 1.0013x over previous
import functools

import jax
import jax.numpy as jnp
import numpy as np
from jax import lax
from jax.experimental import pallas as pl
from jax.experimental.pallas import tpu as pltpu
from jax.experimental.pallas import tpu_sc as plsc

F = 26
VOCAB = 100000
EMB = 16
BATCH = 16384
D1 = 32
D2 = 32

ROWS = BATCH * F
NC, NS = 2, 16
NW = NC * NS
RPW = ROWS // NW
CH = 1664
NCH = RPW // CH
BPC = CH // F
FOUT_RPC = BPC * 32 // 128


CH32 = BPC * 32
CHR = CH // 8


def _gather_body(sec_hbm, first_hbm, idxf_hbm, idx32_hbm,
                 sec_out, first_out,
                 idx_v, rows_v, rv128, idx32_v, f1_v, f2_v, sem, fsem):
    wid = lax.axis_index("s") * NC + lax.axis_index("c")
    base = wid * RPW

    def chunk_body(c, carry):
        off = base + c * CH
        b0 = off // F
        pltpu.sync_copy(idxf_hbm.at[pl.ds(off, CH)], idx_v)
        pltpu.async_copy(sec_hbm.at[idx_v], rows_v, sem).wait()
        for j in range(CHR):
            for k in range(8):
                rv128[j, pl.ds(k * 16, 16)] = rows_v[8 * j + k, :]
        pltpu.sync_copy(rv128, sec_out.at[pl.ds(off // 8, CHR)])

        pltpu.sync_copy(idx32_hbm.at[pl.ds(b0 * 32, CH32)], idx32_v)
        pltpu.async_copy(first_hbm.at[idx32_v], f1_v, fsem).wait()
        for r in range(FOUT_RPC):
            for sg in range(8):
                f2_v[r, pl.ds(sg * 16, 16)] = (
                    f1_v[pl.ds(r * 128 + sg * 16, 16)])
        pltpu.sync_copy(f2_v, first_out.at[pl.ds(b0 // 4, FOUT_RPC)])
        return carry

    lax.fori_loop(0, NCH, chunk_body, 0)


@functools.lru_cache(maxsize=None)
def _make_gather():
    return pl.kernel(
        _gather_body,
        mesh=plsc.VectorSubcoreMesh(core_axis_name="c", subcore_axis_name="s"),
        compiler_params=pltpu.CompilerParams(use_tc_tiling_on_sc=False),
        out_type=(
            jax.ShapeDtypeStruct((ROWS // 8, 128), jnp.float32),
            jax.ShapeDtypeStruct((BATCH * 32 // 128, 128), jnp.float32),
        ),
        scratch_types=[
            pltpu.VMEM((CH,), jnp.int32),
            pltpu.VMEM((CH, EMB), jnp.float32),
            pltpu.VMEM((CHR, 128), jnp.float32),
            pltpu.VMEM((CH32,), jnp.int32),
            pltpu.VMEM((CH32,), jnp.float32),
            pltpu.VMEM((FOUT_RPC, 128), jnp.float32),
            pltpu.SemaphoreType.DMA,
            pltpu.SemaphoreType.DMA,
        ],
    )


BS = 4096
NBLK = BATCH // BS
PR = BS // 8


def _mlp_body(sec8_ref, first4_ref, xv_ref, w1bblk_ref,
              w1ablk_ref, s32_ref, p512_ref, b1_ref, w2_ref, b2_ref,
              w3_ref, b3_ref, out_ref, acc, xv4_s):
    f = pl.program_id(1)

    @pl.when(f == 0)
    def _():
        acc[:, :] = jnp.zeros_like(acc)
        xvwide = jnp.concatenate(
            [xv_ref[:, :], jnp.zeros((BS, 128 - F), jnp.float32)], axis=1)
        xvb = xvwide.reshape(BS // 2, 256).reshape(BS // 4, 512)
        xv4_s[:, :] = jnp.dot(xvb, p512_ref[:, :],
                              preferred_element_type=jnp.float32)

    xv4all = xv4_s[:, :]
    lane128 = lax.broadcasted_iota(jnp.int32, (BS // 4, 128), 1)
    xm = jnp.where(lane128 % 32 == f, xv4all, 0.0)
    u = jnp.dot(xm, s32_ref[:, :],
                preferred_element_type=jnp.float32)
    xvq = u.reshape(PR, 256)
    q = jnp.dot(sec8_ref[:, :], w1bblk_ref[0],
                preferred_element_type=jnp.float32)
    acc[:, :] = acc[:, :] + q * xvq

    @pl.when(f == F - 1)
    def _():
        fo4 = first4_ref[:, :] * xv4all
        c4 = jnp.dot(fo4, w1ablk_ref[:, :],
                     preferred_element_type=jnp.float32)
        first8 = c4.reshape(PR, 256)
        h = jnp.maximum(acc[:, :] + first8 + b1_ref[:, :], 0.0)
        h = jnp.maximum(
            jnp.dot(h, w2_ref[:, :], preferred_element_type=jnp.float32)
            + b2_ref[:, :], 0.0)
        out_ref[:, :] = (
            jnp.dot(h, w3_ref[:, :], preferred_element_type=jnp.float32)
            + b3_ref[:, :])


def _mlp(sec8, first4, xv, w1bblk, w1ablk, s32, p512, b1blk, w2blk, b2blk,
         w3blk, b3blk):
    zero2 = lambda i, f: (0, 0)
    return pl.pallas_call(
        _mlp_body,
        grid=(NBLK, F),
        in_specs=[
            pl.BlockSpec((PR, 128), lambda i, f: (f * NBLK + i, 0)),
            pl.BlockSpec((BS // 4, 128), lambda i, f: (i, 0)),
            pl.BlockSpec((BS, F), lambda i, f: (i, 0)),
            pl.BlockSpec((1, 128, 256), lambda i, f: (f, 0, 0)),
            pl.BlockSpec((128, 128), zero2),
            pl.BlockSpec((128, 128), zero2),
            pl.BlockSpec((512, 128), zero2),
            pl.BlockSpec((1, 256), zero2),
            pl.BlockSpec((256, 256), zero2),
            pl.BlockSpec((1, 256), zero2),
            pl.BlockSpec((256, 8), zero2),
            pl.BlockSpec((1, 8), zero2),
        ],
        out_specs=pl.BlockSpec((PR, 8), lambda i, f: (i, 0)),
        out_shape=jax.ShapeDtypeStruct((BATCH // 8, 8), jnp.float32),
        scratch_shapes=[pltpu.VMEM((PR, 256), jnp.float32),
                        pltpu.VMEM((BS // 4, 128), jnp.float32)],
    )(sec8, first4, xv, w1bblk, w1ablk, s32, p512, b1blk, w2blk, b2blk,
      w3blk, b3blk)


@functools.lru_cache(maxsize=None)
def _gsrc():
    p = np.arange(ROWS)
    return jnp.asarray((p % BATCH) * F + p // BATCH, dtype=jnp.int32)


def kernel(Xi, Xv, fm_bias, first_tables, second_tables, W1, b1, W2, b2, W3, b3):
    xi = Xi[:, :, 0].astype(jnp.int32)
    xv = Xv.astype(jnp.float32)
    foff = jnp.arange(F, dtype=jnp.int32) * VOCAB
    xi_off = (xi + foff[None, :]).reshape(ROWS)
    idx_f = jnp.take(xi_off, _gsrc(), axis=0)
    idx32 = jnp.pad(xi + foff[None, :], ((0, 0), (0, 6))).reshape(
        BATCH * 32)
    sec_flat = second_tables.reshape(F * VOCAB, EMB)
    first_flat = first_tables.reshape(F * VOCAB)

    sec8, first4 = _make_gather()(sec_flat, first_flat, idx_f, idx32)

    eye8 = jnp.eye(8, dtype=jnp.float32)
    s32 = jnp.kron(jnp.eye(4, dtype=jnp.float32),
                   jnp.ones((32, 32), jnp.float32))
    p512np = np.zeros((512, 128), np.float32)
    for k in range(4):
        for j in range(32):
            p512np[k * 128 + j, k * 32 + j] = 1.0
    p512 = jnp.asarray(p512np)
    w1b3 = W1[1 + F:, :].reshape(F, EMB, D1)
    w1bblk = (eye8[None, :, None, :, None]
              * w1b3[:, None, :, None, :]).reshape(F, 128, 8 * D1)
    w1apad = jnp.pad(W1[1:1 + F, :], ((0, 6), (0, 0)))
    w1ablk = jnp.kron(jnp.eye(4, dtype=jnp.float32), w1apad)
    b1e = (b1 + fm_bias * W1[0, :]).reshape(1, D1)
    b1blk = jnp.tile(b1e, (1, 8))
    w2blk = jnp.kron(eye8, W2)
    b2blk = jnp.tile(b2.reshape(1, D2), (1, 8))
    w3blk = jnp.kron(eye8, W3)
    b3blk = jnp.tile(b3.reshape(1, 1), (1, 8))

    out8 = _mlp(sec8, first4, xv, w1bblk, w1ablk, s32, p512, b1blk,
                w2blk, b2blk, w3blk, b3blk)
    return out8.reshape(BATCH)
